# baseline (speedup 1.0000x reference)
import functools

import jax
import jax.numpy as jnp
from jax.experimental import pallas as pl
from jax.experimental.pallas import tpu as pltpu

N = 50000
E = 800000
D_IN = 64
H = 128
EPS = 1e-5
BN = 400


def _proj_body(x_ref, w_ref, b_ref, o_ref):
    o_ref[...] = jax.nn.relu(
        jnp.dot(x_ref[...], w_ref[...], preferred_element_type=jnp.float32)
        + b_ref[...]
    )


def _proj(x, W_in, b_in):
    grid = (N // BN,)
    return pl.pallas_call(
        _proj_body,
        grid=grid,
        in_specs=[
            pl.BlockSpec((BN, D_IN), lambda i: (i, 0)),
            pl.BlockSpec((D_IN, H), lambda i: (0, 0)),
            pl.BlockSpec((1, H), lambda i: (0, 0)),
        ],
        out_specs=pl.BlockSpec((BN, H), lambda i: (i, 0)),
        out_shape=jax.ShapeDtypeStruct((N, H), jnp.float32),
    )(x, W_in, b_in.reshape(1, H))


def _layer_body(h_ref, agg_ref, cnt_ref, wl_ref, bl_ref, wr_ref, g_ref, b_ref,
                o_ref):
    h = h_ref[...]
    inv = 1.0 / jnp.maximum(cnt_ref[...], 1.0)
    mean = agg_ref[...] * inv
    t = (jnp.dot(mean, wl_ref[...], preferred_element_type=jnp.float32)
         + bl_ref[...]
         + jnp.dot(h, wr_ref[...], preferred_element_type=jnp.float32))
    mu = jnp.mean(t, axis=-1, keepdims=True)
    var = jnp.mean((t - mu) ** 2, axis=-1, keepdims=True)
    y = (t - mu) * jax.lax.rsqrt(var + EPS) * g_ref[...] + b_ref[...]
    o_ref[...] = jax.nn.relu(y) + h


def _layer_update(h, agg, cnt, W_l, b_l, W_r, g, b):
    grid = (N // BN,)
    return pl.pallas_call(
        _layer_body,
        grid=grid,
        in_specs=[
            pl.BlockSpec((BN, H), lambda i: (i, 0)),
            pl.BlockSpec((BN, H), lambda i: (i, 0)),
            pl.BlockSpec((BN, 1), lambda i: (i, 0)),
            pl.BlockSpec((H, H), lambda i: (0, 0)),
            pl.BlockSpec((1, H), lambda i: (0, 0)),
            pl.BlockSpec((H, H), lambda i: (0, 0)),
            pl.BlockSpec((1, H), lambda i: (0, 0)),
            pl.BlockSpec((1, H), lambda i: (0, 0)),
        ],
        out_specs=pl.BlockSpec((BN, H), lambda i: (i, 0)),
        out_shape=jax.ShapeDtypeStruct((N, H), jnp.float32),
    )(h, agg, cnt, W_l, b_l.reshape(1, H), W_r, g.reshape(1, H),
      b.reshape(1, H))


def kernel(x, edge_index, W_in, b_in, W_l_0, b_l_0, W_r_0, ln_g_0, ln_b_0,
           W_l_1, b_l_1, W_r_1, ln_g_1, ln_b_1,
           W_l_2, b_l_2, W_r_2, ln_g_2, ln_b_2):
    src = edge_index[0]
    dst = edge_index[1]
    h = _proj(x, W_in, b_in)
    cnt = jax.ops.segment_sum(jnp.ones((E,), jnp.float32), dst,
                              num_segments=N).reshape(N, 1)
    layers = [
        (W_l_0, b_l_0, W_r_0, ln_g_0, ln_b_0),
        (W_l_1, b_l_1, W_r_1, ln_g_1, ln_b_1),
        (W_l_2, b_l_2, W_r_2, ln_g_2, ln_b_2),
    ]
    for (W_l, b_l, W_r, g, b) in layers:
        agg = jax.ops.segment_sum(h[src], dst, num_segments=N)
        h = _layer_update(h, agg, cnt, W_l, b_l, W_r, g, b)
    return h


# speedup vs baseline: 2.6898x; 2.6898x over previous
import functools

import jax
import jax.numpy as jnp
from jax import lax
from jax.experimental import pallas as pl
from jax.experimental.pallas import tpu as pltpu
from jax.experimental.pallas import tpu_sc as plsc

N = 50000
E = 800000
D_IN = 64
H = 128
EPS = 1e-5
BN = 400

NCHUNK = 6
CPC = 3
C = 8448
NPAD = NCHUNK * C
ACC_ROWS = C + 16
RPT = C // 16
NS = 16
SB = 2048
NBLK = 26
EPT = SB * NBLK
E_PAD = NS * EPT
PAD_DST = 1 << 20

_SC_PARAMS = pltpu.CompilerParams(needs_layout_passes=False)


def _proj_body(x_ref, w_ref, b_ref, o_ref):
    o_ref[...] = jax.nn.relu(
        jnp.dot(x_ref[...], w_ref[...], preferred_element_type=jnp.float32)
        + b_ref[...]
    )


def _proj(x, W_in, b_in):
    return pl.pallas_call(
        _proj_body,
        grid=(N // BN,),
        in_specs=[
            pl.BlockSpec((BN, D_IN), lambda i: (i, 0)),
            pl.BlockSpec((D_IN, H), lambda i: (0, 0)),
            pl.BlockSpec((1, H), lambda i: (0, 0)),
        ],
        out_specs=pl.BlockSpec((BN, H), lambda i: (i, 0)),
        out_shape=jax.ShapeDtypeStruct((N, H), jnp.float32),
    )(x, W_in, b_in.reshape(1, H))


def _layer_body(h_ref, agg_ref, cnt_ref, wl_ref, bl_ref, wr_ref, g_ref, b_ref,
                o_ref):
    h = h_ref[...]
    inv = 1.0 / jnp.maximum(cnt_ref[...][:, 0:1], 1.0)
    mean = agg_ref[...] * inv
    t = (jnp.dot(mean, wl_ref[...], preferred_element_type=jnp.float32)
         + bl_ref[...]
         + jnp.dot(h, wr_ref[...], preferred_element_type=jnp.float32))
    mu = jnp.mean(t, axis=-1, keepdims=True)
    var = jnp.mean((t - mu) ** 2, axis=-1, keepdims=True)
    y = (t - mu) * lax.rsqrt(var + EPS) * g_ref[...] + b_ref[...]
    o_ref[...] = jax.nn.relu(y) + h


def _layer_update(h, agg, cnt16, W_l, b_l, W_r, g, b):
    return pl.pallas_call(
        _layer_body,
        grid=(N // BN,),
        in_specs=[
            pl.BlockSpec((BN, H), lambda i: (i, 0)),
            pl.BlockSpec((BN, H), lambda i: (i, 0)),
            pl.BlockSpec((BN, H), lambda i: (i, 0)),
            pl.BlockSpec((H, H), lambda i: (0, 0)),
            pl.BlockSpec((1, H), lambda i: (0, 0)),
            pl.BlockSpec((H, H), lambda i: (0, 0)),
            pl.BlockSpec((1, H), lambda i: (0, 0)),
            pl.BlockSpec((1, H), lambda i: (0, 0)),
        ],
        out_specs=pl.BlockSpec((BN, H), lambda i: (i, 0)),
        out_shape=jax.ShapeDtypeStruct((N, H), jnp.float32),
    )(h, agg, cnt16, W_l, b_l.reshape(1, H), W_r, g.reshape(1, H),
      b.reshape(1, H))


_SC_MESH = plsc.VectorSubcoreMesh(core_axis_name="c", subcore_axis_name="s")


def _compact_scan(edt, est, srcp, dstp, comp_s, comp_d, ebase, lo, flush):
    def vec_body(i, count):
        dv = edt[pl.ds(i * 16, 16)]
        dl = dv - lo
        m = (dl >= 0) & (dl < C)
        mi = m.astype(jnp.int32)
        pos = count + plsc.cumsum(mi) - 1
        if comp_s is not None:
            sv = est[pl.ds(i * 16, 16)]
            plsc.store_scatter(comp_s, [pos], sv, mask=m)
        plsc.store_scatter(comp_d, [pos], dl, mask=m)
        count = count + jnp.sum(mi)
        return lax.cond(count >= 128, flush, lambda c: c, count)

    def blk_body(b, count):
        if comp_s is not None:
            pltpu.sync_copy(srcp.at[pl.ds(ebase + b * SB, SB)], est)
        pltpu.sync_copy(dstp.at[pl.ds(ebase + b * SB, SB)], edt)
        return lax.fori_loop(0, SB // 16, vec_body, count)

    return lax.fori_loop(0, NBLK, blk_body, jnp.int32(0))


def _pad_tail(comp, xfer, count, fill):
    iot = lax.iota(jnp.int32, 16)
    for k in range(8):
        m = (iot + (k * 16)) < count
        xfer[pl.ds(k * 16, 16)] = jnp.where(m, comp[pl.ds(k * 16, 16)], fill)


def _segsum_body(h, srcp, dstp, z128, agg, *, acc, est, edt, comp_s, comp_d,
                 xfer_s, xfer_d, rowbuf, sem):
    cid = lax.axis_index("c")
    sid = lax.axis_index("s")
    ebase = sid * EPT

    def flush(count):
        for k in range(8):
            xfer_s[pl.ds(k * 16, 16)] = comp_s[pl.ds(k * 16, 16)]
            xfer_d[pl.ds(k * 16, 16)] = comp_d[pl.ds(k * 16, 16)]
        pltpu.async_copy(h.at[xfer_s], rowbuf, sem).wait()
        pltpu.sync_copy(rowbuf, acc.at[xfer_d], add=True)
        for k in range(8):
            comp_s[pl.ds(k * 16, 16)] = comp_s[pl.ds(128 + k * 16, 16)]
            comp_d[pl.ds(k * 16, 16)] = comp_d[pl.ds(128 + k * 16, 16)]
        return count - 128

    for cc in range(CPC):
        chunk = cid * CPC + cc
        lo = chunk * C
        pltpu.sync_copy(z128, acc.at[pl.ds(sid * RPT, RPT)])
        plsc.subcore_barrier()

        count = _compact_scan(edt, est, srcp, dstp, comp_s, comp_d, ebase,
                              lo, flush)

        _pad_tail(comp_s, xfer_s, count, 0)
        _pad_tail(comp_d, xfer_d, count, C)
        pltpu.async_copy(h.at[xfer_s], rowbuf, sem).wait()
        pltpu.sync_copy(rowbuf, acc.at[xfer_d], add=True)
        plsc.subcore_barrier()

        pltpu.sync_copy(acc.at[pl.ds(sid * RPT, RPT)],
                        agg.at[pl.ds(chunk * C + sid * RPT, RPT)])
        plsc.subcore_barrier()


_segsum = pl.kernel(
    _segsum_body,
    out_type=(jax.ShapeDtypeStruct((NPAD, H), jnp.float32),),
    mesh=_SC_MESH,
    scratch_types=dict(
        acc=pltpu.VMEM_SHARED((ACC_ROWS, H), jnp.float32),
        est=pltpu.VMEM((SB,), jnp.int32),
        edt=pltpu.VMEM((SB,), jnp.int32),
        comp_s=pltpu.VMEM((256,), jnp.int32),
        comp_d=pltpu.VMEM((256,), jnp.int32),
        xfer_s=pltpu.VMEM((128,), jnp.int32),
        xfer_d=pltpu.VMEM((128,), jnp.int32),
        rowbuf=pltpu.VMEM((128, H), jnp.float32),
        sem=pltpu.SemaphoreType.DMA,
    ),
    compiler_params=_SC_PARAMS,
)


def kernel(x, edge_index, W_in, b_in, W_l_0, b_l_0, W_r_0, ln_g_0, ln_b_0,
           W_l_1, b_l_1, W_r_1, ln_g_1, ln_b_1,
           W_l_2, b_l_2, W_r_2, ln_g_2, ln_b_2):
    src = edge_index[0]
    dst = edge_index[1]
    srcp = jnp.concatenate([src, jnp.zeros((E_PAD - E,), jnp.int32)])
    dstp = jnp.concatenate(
        [dst, jnp.full((E_PAD - E,), PAD_DST, jnp.int32)])
    ones16 = jnp.ones((128, 16), jnp.float32)
    z128 = jnp.zeros((RPT, H), jnp.float32)
    z16 = jnp.zeros((RPT, 16), jnp.float32)

    h = _proj(x, W_in, b_in)
    ones_tab = jnp.ones((N, H), jnp.float32)
    (cnt16,) = _segsum(ones_tab, srcp, dstp, z128)
    layers = [
        (W_l_0, b_l_0, W_r_0, ln_g_0, ln_b_0),
        (W_l_1, b_l_1, W_r_1, ln_g_1, ln_b_1),
        (W_l_2, b_l_2, W_r_2, ln_g_2, ln_b_2),
    ]
    for (W_l, b_l, W_r, g, b) in layers:
        (agg,) = _segsum(h, srcp, dstp, z128)
        h = _layer_update(h, agg, cnt16, W_l, b_l, W_r, g, b)
    return h


# speedup vs baseline: 4.0858x; 1.5190x over previous
import functools

import jax
import jax.numpy as jnp
from jax import lax
from jax.experimental import pallas as pl
from jax.experimental.pallas import tpu as pltpu
from jax.experimental.pallas import tpu_sc as plsc

N = 50000
E = 800000
D_IN = 64
H = 128
EPS = 1e-5
BN = 400

NCHUNK = 6
CPC = 3
C = 8448
NPAD = NCHUNK * C
ACC_ROWS = C + 16
RPT = C // 16
NS = 16
SB = 2048
NBLK = 26
EPT = SB * NBLK
E_PAD = NS * EPT
PAD_DST = 1 << 20
SUP = 2048
CAP = EPT + SUP
NBUCKET = NCHUNK * NS

_SC_PARAMS = pltpu.CompilerParams(needs_layout_passes=False)
_SC_MESH = plsc.VectorSubcoreMesh(core_axis_name="c", subcore_axis_name="s")


def _proj_body(x_ref, w_ref, b_ref, o_ref):
    o_ref[...] = jax.nn.relu(
        jnp.dot(x_ref[...], w_ref[...], preferred_element_type=jnp.float32)
        + b_ref[...]
    )


def _proj(x, W_in, b_in):
    return pl.pallas_call(
        _proj_body,
        grid=(N // BN,),
        in_specs=[
            pl.BlockSpec((BN, D_IN), lambda i: (i, 0)),
            pl.BlockSpec((D_IN, H), lambda i: (0, 0)),
            pl.BlockSpec((1, H), lambda i: (0, 0)),
        ],
        out_specs=pl.BlockSpec((BN, H), lambda i: (i, 0)),
        out_shape=jax.ShapeDtypeStruct((N, H), jnp.float32),
    )(x, W_in, b_in.reshape(1, H))


def _layer_body(h_ref, agg_ref, cnt_ref, wl_ref, bl_ref, wr_ref, g_ref, b_ref,
                o_ref):
    h = h_ref[...]
    inv = 1.0 / jnp.maximum(cnt_ref[...][:, 0:1], 1.0)
    mean = agg_ref[...] * inv
    t = (jnp.dot(mean, wl_ref[...], preferred_element_type=jnp.float32)
         + bl_ref[...]
         + jnp.dot(h, wr_ref[...], preferred_element_type=jnp.float32))
    mu = jnp.mean(t, axis=-1, keepdims=True)
    var = jnp.mean((t - mu) ** 2, axis=-1, keepdims=True)
    y = (t - mu) * lax.rsqrt(var + EPS) * g_ref[...] + b_ref[...]
    o_ref[...] = jax.nn.relu(y) + h


def _layer_update(h, agg, cnt, W_l, b_l, W_r, g, b):
    return pl.pallas_call(
        _layer_body,
        grid=(N // BN,),
        in_specs=[
            pl.BlockSpec((BN, H), lambda i: (i, 0)),
            pl.BlockSpec((BN, H), lambda i: (i, 0)),
            pl.BlockSpec((BN, H), lambda i: (i, 0)),
            pl.BlockSpec((H, H), lambda i: (0, 0)),
            pl.BlockSpec((1, H), lambda i: (0, 0)),
            pl.BlockSpec((H, H), lambda i: (0, 0)),
            pl.BlockSpec((1, H), lambda i: (0, 0)),
            pl.BlockSpec((1, H), lambda i: (0, 0)),
        ],
        out_specs=pl.BlockSpec((BN, H), lambda i: (i, 0)),
        out_shape=jax.ShapeDtypeStruct((N, H), jnp.float32),
    )(h, agg, cnt, W_l, b_l.reshape(1, H), W_r, g.reshape(1, H),
      b.reshape(1, H))


def _route_body(srcp, dstp, lsrc, ldst, nblk1d, *, est, edt, comp_s, comp_d,
                nbuf):
    cid = lax.axis_index("c")
    sid = lax.axis_index("s")
    ebase = sid * EPT

    for cc in range(CPC):
        chunk = cid * CPC + cc
        lo = chunk * C
        bucket = chunk * NS + sid
        base = bucket * CAP

        def flush(carry):
            count, ws = carry
            pltpu.sync_copy(comp_s.at[pl.ds(0, SUP)],
                            lsrc.at[pl.ds(base + ws * SUP, SUP)])
            pltpu.sync_copy(comp_d.at[pl.ds(0, SUP)],
                            ldst.at[pl.ds(base + ws * SUP, SUP)])
            comp_s[pl.ds(0, 16)] = comp_s[pl.ds(SUP, 16)]
            comp_d[pl.ds(0, 16)] = comp_d[pl.ds(SUP, 16)]
            return count - SUP, ws + 1

        def vec_body(i, carry):
            count, ws = carry
            dv = edt[pl.ds(i * 16, 16)]
            sv = est[pl.ds(i * 16, 16)]
            dl = dv - lo
            m = (dl >= 0) & (dl < C)
            mi = m.astype(jnp.int32)
            pos = count + plsc.cumsum(mi) - 1
            plsc.store_scatter(comp_s, [pos], sv, mask=m)
            plsc.store_scatter(comp_d, [pos], dl, mask=m)
            count = count + jnp.sum(mi)
            return lax.cond(count >= SUP, flush, lambda c: c, (count, ws))

        def blk_body(b, carry):
            pltpu.sync_copy(srcp.at[pl.ds(ebase + b * SB, SB)], est)
            pltpu.sync_copy(dstp.at[pl.ds(ebase + b * SB, SB)], edt)
            return lax.fori_loop(0, SB // 16, vec_body, carry)

        count, ws = lax.fori_loop(0, NBLK, blk_body,
                                  (jnp.int32(0), jnp.int32(0)))

        iot = lax.iota(jnp.int32, 16)

        def pad_body(j, _):
            off = j * 16
            m = (iot + off) < count
            comp_s[pl.ds(off, 16)] = jnp.where(m, comp_s[pl.ds(off, 16)], 0)
            comp_d[pl.ds(off, 16)] = jnp.where(m, comp_d[pl.ds(off, 16)], C)
            return 0

        lax.fori_loop(0, SUP // 16, pad_body, 0)
        pltpu.sync_copy(comp_s.at[pl.ds(0, SUP)],
                        lsrc.at[pl.ds(base + ws * SUP, SUP)])
        pltpu.sync_copy(comp_d.at[pl.ds(0, SUP)],
                        ldst.at[pl.ds(base + ws * SUP, SUP)])

        nblk = ws * 16 + lax.div(count + 127, 128)
        nbuf[...] = jnp.zeros((16,), jnp.int32) + nblk
        pltpu.sync_copy(nbuf, nblk1d.at[pl.ds(bucket * 16, 16)])


_route = pl.kernel(
    _route_body,
    out_type=(
        jax.ShapeDtypeStruct((NBUCKET * CAP,), jnp.int32),
        jax.ShapeDtypeStruct((NBUCKET * CAP,), jnp.int32),
        jax.ShapeDtypeStruct((NBUCKET * 16,), jnp.int32),
    ),
    mesh=_SC_MESH,
    scratch_types=dict(
        est=pltpu.VMEM((SB,), jnp.int32),
        edt=pltpu.VMEM((SB,), jnp.int32),
        comp_s=pltpu.VMEM((SUP + 128,), jnp.int32),
        comp_d=pltpu.VMEM((SUP + 128,), jnp.int32),
        nbuf=pltpu.VMEM((16,), jnp.int32),
    ),
    compiler_params=_SC_PARAMS,
)


def _stream_body(h, lsrc, ldst, nblk1d, z128, agg, *, acc, sbuf_s, sbuf_d,
                 isrc0, isrc1, idst0, idst1, rowbuf0, rowbuf1, nbuf,
                 semg0, semg1):
    cid = lax.axis_index("c")
    sid = lax.axis_index("s")

    def load_idx(kk, isrc, idst):
        for j in range(8):
            isrc[pl.ds(j * 16, 16)] = sbuf_s[pl.ds(kk * 128 + j * 16, 16)]
            idst[pl.ds(j * 16, 16)] = sbuf_d[pl.ds(kk * 128 + j * 16, 16)]

    for cc in range(CPC):
        chunk = cid * CPC + cc
        bucket = chunk * NS + sid
        base = bucket * CAP

        pltpu.sync_copy(z128, acc.at[pl.ds(sid * RPT, RPT)])
        plsc.subcore_barrier()

        pltpu.sync_copy(nblk1d.at[pl.ds(bucket * 16, 16)], nbuf)
        nb = jnp.max(nbuf[...])
        ns = lax.div(nb + 15, 16)

        def sup_body(s, _):
            pltpu.sync_copy(lsrc.at[pl.ds(base + s * SUP, SUP)], sbuf_s)
            pltpu.sync_copy(ldst.at[pl.ds(base + s * SUP, SUP)], sbuf_d)
            nbk = jnp.minimum(16, nb - s * 16)

            load_idx(0, isrc0, idst0)
            pltpu.async_copy(h.at[isrc0], rowbuf0, semg0)

            def kb(k, _):
                even = lax.rem(k, 2) == 0

                @pl.when(even)
                def _():
                    @pl.when(k + 1 < nbk)
                    def _():
                        load_idx(k + 1, isrc1, idst1)
                        pltpu.async_copy(h.at[isrc1], rowbuf1, semg1)
                    pltpu.make_async_copy(h.at[isrc0], rowbuf0, semg0).wait()
                    pltpu.sync_copy(rowbuf0, acc.at[idst0], add=True)

                @pl.when(jnp.logical_not(even))
                def _():
                    @pl.when(k + 1 < nbk)
                    def _():
                        load_idx(k + 1, isrc0, idst0)
                        pltpu.async_copy(h.at[isrc0], rowbuf0, semg0)
                    pltpu.make_async_copy(h.at[isrc1], rowbuf1, semg1).wait()
                    pltpu.sync_copy(rowbuf1, acc.at[idst1], add=True)

                return 0

            lax.fori_loop(0, nbk, kb, 0)
            return 0

        lax.fori_loop(0, ns, sup_body, 0)
        plsc.subcore_barrier()

        pltpu.sync_copy(acc.at[pl.ds(sid * RPT, RPT)],
                        agg.at[pl.ds(chunk * C + sid * RPT, RPT)])
        plsc.subcore_barrier()


_stream = pl.kernel(
    _stream_body,
    out_type=(jax.ShapeDtypeStruct((NPAD, H), jnp.float32),),
    mesh=_SC_MESH,
    scratch_types=dict(
        acc=pltpu.VMEM_SHARED((ACC_ROWS, H), jnp.float32),
        sbuf_s=pltpu.VMEM((SUP,), jnp.int32),
        sbuf_d=pltpu.VMEM((SUP,), jnp.int32),
        isrc0=pltpu.VMEM((128,), jnp.int32),
        isrc1=pltpu.VMEM((128,), jnp.int32),
        idst0=pltpu.VMEM((128,), jnp.int32),
        idst1=pltpu.VMEM((128,), jnp.int32),
        rowbuf0=pltpu.VMEM((128, H), jnp.float32),
        rowbuf1=pltpu.VMEM((128, H), jnp.float32),
        nbuf=pltpu.VMEM((16,), jnp.int32),
        semg0=pltpu.SemaphoreType.DMA,
        semg1=pltpu.SemaphoreType.DMA,
    ),
    compiler_params=_SC_PARAMS,
)


def kernel(x, edge_index, W_in, b_in, W_l_0, b_l_0, W_r_0, ln_g_0, ln_b_0,
           W_l_1, b_l_1, W_r_1, ln_g_1, ln_b_1,
           W_l_2, b_l_2, W_r_2, ln_g_2, ln_b_2):
    src = edge_index[0]
    dst = edge_index[1]
    srcp = jnp.concatenate([src, jnp.zeros((E_PAD - E,), jnp.int32)])
    dstp = jnp.concatenate(
        [dst, jnp.full((E_PAD - E,), PAD_DST, jnp.int32)])
    z128 = jnp.zeros((RPT, H), jnp.float32)

    h = _proj(x, W_in, b_in)
    lsrc, ldst, nblk1d = _route(srcp, dstp)
    ones_tab = jnp.ones((N, H), jnp.float32)
    (cnt,) = _stream(ones_tab, lsrc, ldst, nblk1d, z128)
    layers = [
        (W_l_0, b_l_0, W_r_0, ln_g_0, ln_b_0),
        (W_l_1, b_l_1, W_r_1, ln_g_1, ln_b_1),
        (W_l_2, b_l_2, W_r_2, ln_g_2, ln_b_2),
    ]
    for (W_l, b_l, W_r, g, b) in layers:
        (agg,) = _stream(h, lsrc, ldst, nblk1d, z128)
        h = _layer_update(h, agg, cnt, W_l, b_l, W_r, g, b)
    return h


# speedup vs baseline: 4.1232x; 1.0092x over previous
import functools

import jax
import jax.numpy as jnp
from jax import lax
from jax.experimental import pallas as pl
from jax.experimental.pallas import tpu as pltpu
from jax.experimental.pallas import tpu_sc as plsc

N = 50000
E = 800000
D_IN = 64
H = 128
EPS = 1e-5
BN = 400

NCHUNK = 6
CPC = 3
C = 8448
NPAD = NCHUNK * C
ACC_ROWS = C + 16
RPT = C // 16
NS = 16
SB = 2048
NBLK = 26
EPT = SB * NBLK
E_PAD = NS * EPT
PAD_DST = 1 << 20
SUP = 2048
CAP = EPT + SUP
NBUCKET = NCHUNK * NS

_SC_PARAMS = pltpu.CompilerParams(needs_layout_passes=False)
_SC_MESH = plsc.VectorSubcoreMesh(core_axis_name="c", subcore_axis_name="s")


def _proj_body(x_ref, w_ref, b_ref, o_ref):
    o_ref[...] = jax.nn.relu(
        jnp.dot(x_ref[...], w_ref[...], preferred_element_type=jnp.float32)
        + b_ref[...]
    )


def _proj(x, W_in, b_in):
    return pl.pallas_call(
        _proj_body,
        grid=(N // BN,),
        in_specs=[
            pl.BlockSpec((BN, D_IN), lambda i: (i, 0)),
            pl.BlockSpec((D_IN, H), lambda i: (0, 0)),
            pl.BlockSpec((1, H), lambda i: (0, 0)),
        ],
        out_specs=pl.BlockSpec((BN, H), lambda i: (i, 0)),
        out_shape=jax.ShapeDtypeStruct((N, H), jnp.float32),
    )(x, W_in, b_in.reshape(1, H))


def _layer_body(h_ref, agg_ref, cnt_ref, wl_ref, bl_ref, wr_ref, g_ref, b_ref,
                o_ref):
    h = h_ref[...]
    inv = 1.0 / jnp.maximum(cnt_ref[...][:, 0:1], 1.0)
    mean = agg_ref[...] * inv
    t = (jnp.dot(mean, wl_ref[...], preferred_element_type=jnp.float32)
         + bl_ref[...]
         + jnp.dot(h, wr_ref[...], preferred_element_type=jnp.float32))
    mu = jnp.mean(t, axis=-1, keepdims=True)
    var = jnp.mean((t - mu) ** 2, axis=-1, keepdims=True)
    y = (t - mu) * lax.rsqrt(var + EPS) * g_ref[...] + b_ref[...]
    o_ref[...] = jax.nn.relu(y) + h


def _layer_update(h, agg, cnt, W_l, b_l, W_r, g, b):
    return pl.pallas_call(
        _layer_body,
        grid=(N // BN,),
        in_specs=[
            pl.BlockSpec((BN, H), lambda i: (i, 0)),
            pl.BlockSpec((BN, H), lambda i: (i, 0)),
            pl.BlockSpec((BN, H), lambda i: (i, 0)),
            pl.BlockSpec((H, H), lambda i: (0, 0)),
            pl.BlockSpec((1, H), lambda i: (0, 0)),
            pl.BlockSpec((H, H), lambda i: (0, 0)),
            pl.BlockSpec((1, H), lambda i: (0, 0)),
            pl.BlockSpec((1, H), lambda i: (0, 0)),
        ],
        out_specs=pl.BlockSpec((BN, H), lambda i: (i, 0)),
        out_shape=jax.ShapeDtypeStruct((N, H), jnp.float32),
    )(h, agg, cnt, W_l, b_l.reshape(1, H), W_r, g.reshape(1, H),
      b.reshape(1, H))


def _route_body(srcp, dstp, lsrc, ldst, nblk1d, *, est, edt, comp_s, comp_d,
                nbuf):
    cid = lax.axis_index("c")
    sid = lax.axis_index("s")
    ebase = sid * EPT

    for cc in range(CPC):
        chunk = cid * CPC + cc
        lo = chunk * C
        bucket = chunk * NS + sid
        base = bucket * CAP

        def flush(carry):
            count, ws = carry
            pltpu.sync_copy(comp_s.at[pl.ds(0, SUP)],
                            lsrc.at[pl.ds(base + ws * SUP, SUP)])
            pltpu.sync_copy(comp_d.at[pl.ds(0, SUP)],
                            ldst.at[pl.ds(base + ws * SUP, SUP)])
            comp_s[pl.ds(0, 16)] = comp_s[pl.ds(SUP, 16)]
            comp_d[pl.ds(0, 16)] = comp_d[pl.ds(SUP, 16)]
            return count - SUP, ws + 1

        def vec_body(i, carry):
            count, ws = carry
            dv = edt[pl.ds(i * 16, 16)]
            sv = est[pl.ds(i * 16, 16)]
            dl = dv - lo
            m = (dl >= 0) & (dl < C)
            mi = m.astype(jnp.int32)
            pos = count + plsc.cumsum(mi) - 1
            plsc.store_scatter(comp_s, [pos], sv, mask=m)
            plsc.store_scatter(comp_d, [pos], dl, mask=m)
            count = count + jnp.sum(mi)
            return lax.cond(count >= SUP, flush, lambda c: c, (count, ws))

        def blk_body(b, carry):
            pltpu.sync_copy(srcp.at[pl.ds(ebase + b * SB, SB)], est)
            pltpu.sync_copy(dstp.at[pl.ds(ebase + b * SB, SB)], edt)
            return lax.fori_loop(0, SB // 16, vec_body, carry)

        count, ws = lax.fori_loop(0, NBLK, blk_body,
                                  (jnp.int32(0), jnp.int32(0)))

        iot = lax.iota(jnp.int32, 16)

        def pad_body(j, _):
            off = j * 16
            m = (iot + off) < count
            comp_s[pl.ds(off, 16)] = jnp.where(m, comp_s[pl.ds(off, 16)], 0)
            comp_d[pl.ds(off, 16)] = jnp.where(m, comp_d[pl.ds(off, 16)], C)
            return 0

        lax.fori_loop(0, SUP // 16, pad_body, 0)
        pltpu.sync_copy(comp_s.at[pl.ds(0, SUP)],
                        lsrc.at[pl.ds(base + ws * SUP, SUP)])
        pltpu.sync_copy(comp_d.at[pl.ds(0, SUP)],
                        ldst.at[pl.ds(base + ws * SUP, SUP)])

        nblk = ws * 16 + lax.div(count + 127, 128)
        nbuf[...] = jnp.zeros((16,), jnp.int32) + nblk
        pltpu.sync_copy(nbuf, nblk1d.at[pl.ds(bucket * 16, 16)])


_route = pl.kernel(
    _route_body,
    out_type=(
        jax.ShapeDtypeStruct((NBUCKET * CAP,), jnp.int32),
        jax.ShapeDtypeStruct((NBUCKET * CAP,), jnp.int32),
        jax.ShapeDtypeStruct((NBUCKET * 16,), jnp.int32),
    ),
    mesh=_SC_MESH,
    scratch_types=dict(
        est=pltpu.VMEM((SB,), jnp.int32),
        edt=pltpu.VMEM((SB,), jnp.int32),
        comp_s=pltpu.VMEM((SUP + 128,), jnp.int32),
        comp_d=pltpu.VMEM((SUP + 128,), jnp.int32),
        nbuf=pltpu.VMEM((16,), jnp.int32),
    ),
    compiler_params=_SC_PARAMS,
)


def _stream_body(h, lsrc, ldst, nblk1d, z128, agg, *, acc, sbuf_s, sbuf_d,
                 idst0, idst1, rowbuf0, rowbuf1, nbuf,
                 semg0, semg1, sema0, sema1):
    cid = lax.axis_index("c")
    sid = lax.axis_index("s")

    def load_idst(kk, idst):
        for j in range(8):
            idst[pl.ds(j * 16, 16)] = sbuf_d[pl.ds(kk * 128 + j * 16, 16)]

    def gat(kk, rowbuf, semg):
        pltpu.async_copy(h.at[sbuf_s.at[pl.ds(kk * 128, 128)]], rowbuf, semg)

    for cc in range(CPC):
        chunk = cid * CPC + cc
        bucket = chunk * NS + sid
        base = bucket * CAP

        pltpu.sync_copy(z128, acc.at[pl.ds(sid * RPT, RPT)])
        plsc.subcore_barrier()

        pltpu.sync_copy(nblk1d.at[pl.ds(bucket * 16, 16)], nbuf)
        nb = jnp.max(nbuf[...])
        ns = lax.div(nb + 15, 16)

        def sup_body(s, _):
            pltpu.sync_copy(lsrc.at[pl.ds(base + s * SUP, SUP)], sbuf_s)
            pltpu.sync_copy(ldst.at[pl.ds(base + s * SUP, SUP)], sbuf_d)
            nbk = jnp.minimum(16, nb - s * 16)

            load_idst(0, idst0)
            gat(0, rowbuf0, semg0)

            def kb(k, _):
                even = lax.rem(k, 2) == 0

                @pl.when(even)
                def _():
                    @pl.when(k + 1 < nbk)
                    def _():
                        @pl.when(k >= 1)
                        def _():
                            pltpu.make_async_copy(
                                rowbuf1, acc.at[idst1], sema1).wait()
                        load_idst(k + 1, idst1)
                        gat(k + 1, rowbuf1, semg1)
                    pltpu.make_async_copy(
                        h.at[sbuf_s.at[pl.ds(k * 128, 128)]], rowbuf0,
                        semg0).wait()
                    pltpu.async_copy(rowbuf0, acc.at[idst0], sema0, add=True)

                @pl.when(jnp.logical_not(even))
                def _():
                    @pl.when(k + 1 < nbk)
                    def _():
                        pltpu.make_async_copy(
                            rowbuf0, acc.at[idst0], sema0).wait()
                        load_idst(k + 1, idst0)
                        gat(k + 1, rowbuf0, semg0)
                    pltpu.make_async_copy(
                        h.at[sbuf_s.at[pl.ds(k * 128, 128)]], rowbuf1,
                        semg1).wait()
                    pltpu.async_copy(rowbuf1, acc.at[idst1], sema1, add=True)

                return 0

            lax.fori_loop(0, nbk, kb, 0)

            last_even = lax.rem(nbk - 1, 2) == 0

            @pl.when(last_even)
            def _():
                pltpu.make_async_copy(rowbuf0, acc.at[idst0], sema0).wait()

                @pl.when(nbk > 1)
                def _():
                    pltpu.make_async_copy(rowbuf1, acc.at[idst1], sema1).wait()

            @pl.when(jnp.logical_not(last_even))
            def _():
                pltpu.make_async_copy(rowbuf1, acc.at[idst1], sema1).wait()
                pltpu.make_async_copy(rowbuf0, acc.at[idst0], sema0).wait()

            return 0

        lax.fori_loop(0, ns, sup_body, 0)
        plsc.subcore_barrier()

        pltpu.sync_copy(acc.at[pl.ds(sid * RPT, RPT)],
                        agg.at[pl.ds(chunk * C + sid * RPT, RPT)])
        plsc.subcore_barrier()


_stream = pl.kernel(
    _stream_body,
    out_type=(jax.ShapeDtypeStruct((NPAD, H), jnp.float32),),
    mesh=_SC_MESH,
    scratch_types=dict(
        acc=pltpu.VMEM_SHARED((ACC_ROWS, H), jnp.float32),
        sbuf_s=pltpu.VMEM((SUP,), jnp.int32),
        sbuf_d=pltpu.VMEM((SUP,), jnp.int32),
        idst0=pltpu.VMEM((128,), jnp.int32),
        idst1=pltpu.VMEM((128,), jnp.int32),
        rowbuf0=pltpu.VMEM((128, H), jnp.float32),
        rowbuf1=pltpu.VMEM((128, H), jnp.float32),
        nbuf=pltpu.VMEM((16,), jnp.int32),
        semg0=pltpu.SemaphoreType.DMA,
        semg1=pltpu.SemaphoreType.DMA,
        sema0=pltpu.SemaphoreType.DMA,
        sema1=pltpu.SemaphoreType.DMA,
    ),
    compiler_params=_SC_PARAMS,
)


def kernel(x, edge_index, W_in, b_in, W_l_0, b_l_0, W_r_0, ln_g_0, ln_b_0,
           W_l_1, b_l_1, W_r_1, ln_g_1, ln_b_1,
           W_l_2, b_l_2, W_r_2, ln_g_2, ln_b_2):
    src = edge_index[0]
    dst = edge_index[1]
    srcp = jnp.concatenate([src, jnp.zeros((E_PAD - E,), jnp.int32)])
    dstp = jnp.concatenate(
        [dst, jnp.full((E_PAD - E,), PAD_DST, jnp.int32)])
    z128 = jnp.zeros((RPT, H), jnp.float32)

    h = _proj(x, W_in, b_in)
    lsrc, ldst, nblk1d = _route(srcp, dstp)
    ones_tab = jnp.ones((N, H), jnp.float32)
    (cnt,) = _stream(ones_tab, lsrc, ldst, nblk1d, z128)
    layers = [
        (W_l_0, b_l_0, W_r_0, ln_g_0, ln_b_0),
        (W_l_1, b_l_1, W_r_1, ln_g_1, ln_b_1),
        (W_l_2, b_l_2, W_r_2, ln_g_2, ln_b_2),
    ]
    for (W_l, b_l, W_r, g, b) in layers:
        (agg,) = _stream(h, lsrc, ldst, nblk1d, z128)
        h = _layer_update(h, agg, cnt, W_l, b_l, W_r, g, b)
    return h


# speedup vs baseline: 4.6940x; 1.1384x over previous
import functools

import jax
import jax.numpy as jnp
from jax import lax
from jax.experimental import pallas as pl
from jax.experimental.pallas import tpu as pltpu
from jax.experimental.pallas import tpu_sc as plsc

N = 50000
E = 800000
D_IN = 64
H = 128
EPS = 1e-5
BN = 400

NCHUNK = 6
CPC = 3
C = 8448
NPAD = NCHUNK * C
ACC_ROWS = C + 16
RPT = C // 16
NS = 16
SB = 2048
NBLK = 26
EPT = SB * NBLK
E_PAD = NS * EPT
PAD_DST = 1 << 20
SUP = 2048
CAP = EPT + SUP
NBUCKET = NCHUNK * NS

_SC_PARAMS = pltpu.CompilerParams(needs_layout_passes=False)
_SC_MESH = plsc.VectorSubcoreMesh(core_axis_name="c", subcore_axis_name="s")


def _proj_body(x_ref, w_ref, b_ref, o_ref):
    o_ref[...] = jax.nn.relu(
        jnp.dot(x_ref[...], w_ref[...], preferred_element_type=jnp.float32)
        + b_ref[...]
    )


def _proj(x, W_in, b_in):
    return pl.pallas_call(
        _proj_body,
        grid=(N // BN,),
        in_specs=[
            pl.BlockSpec((BN, D_IN), lambda i: (i, 0)),
            pl.BlockSpec((D_IN, H), lambda i: (0, 0)),
            pl.BlockSpec((1, H), lambda i: (0, 0)),
        ],
        out_specs=pl.BlockSpec((BN, H), lambda i: (i, 0)),
        out_shape=jax.ShapeDtypeStruct((N, H), jnp.float32),
    )(x, W_in, b_in.reshape(1, H))


def _layer_body(h_ref, agg_ref, cnt_ref, wl_ref, bl_ref, wr_ref, g_ref, b_ref,
                o_ref):
    h = h_ref[...]
    inv = 1.0 / jnp.maximum(cnt_ref[...][:, 0:1], 1.0)
    mean = agg_ref[...] * inv
    t = (jnp.dot(mean, wl_ref[...], preferred_element_type=jnp.float32)
         + bl_ref[...]
         + jnp.dot(h, wr_ref[...], preferred_element_type=jnp.float32))
    mu = jnp.mean(t, axis=-1, keepdims=True)
    var = jnp.mean((t - mu) ** 2, axis=-1, keepdims=True)
    y = (t - mu) * lax.rsqrt(var + EPS) * g_ref[...] + b_ref[...]
    o_ref[...] = jax.nn.relu(y) + h


def _layer_update(h, agg, cnt, W_l, b_l, W_r, g, b):
    return pl.pallas_call(
        _layer_body,
        grid=(N // BN,),
        in_specs=[
            pl.BlockSpec((BN, H), lambda i: (i, 0)),
            pl.BlockSpec((BN, H), lambda i: (i, 0)),
            pl.BlockSpec((BN, H), lambda i: (i, 0)),
            pl.BlockSpec((H, H), lambda i: (0, 0)),
            pl.BlockSpec((1, H), lambda i: (0, 0)),
            pl.BlockSpec((H, H), lambda i: (0, 0)),
            pl.BlockSpec((1, H), lambda i: (0, 0)),
            pl.BlockSpec((1, H), lambda i: (0, 0)),
        ],
        out_specs=pl.BlockSpec((BN, H), lambda i: (i, 0)),
        out_shape=jax.ShapeDtypeStruct((N, H), jnp.float32),
    )(h, agg, cnt, W_l, b_l.reshape(1, H), W_r, g.reshape(1, H),
      b.reshape(1, H))


def _route_body(srcp, dstp, lsrc, ldst, nblk1d, *, est, edt, comp_s, comp_d,
                nbuf):
    cid = lax.axis_index("c")
    sid = lax.axis_index("s")
    ebase = sid * EPT

    for cc in range(CPC):
        chunk = cid * CPC + cc
        lo = chunk * C
        bucket = chunk * NS + sid
        base = bucket * CAP

        def flush(carry):
            count, ws = carry
            pltpu.sync_copy(comp_s.at[pl.ds(0, SUP)],
                            lsrc.at[pl.ds(base + ws * SUP, SUP)])
            pltpu.sync_copy(comp_d.at[pl.ds(0, SUP)],
                            ldst.at[pl.ds(base + ws * SUP, SUP)])
            comp_s[pl.ds(0, 16)] = comp_s[pl.ds(SUP, 16)]
            comp_d[pl.ds(0, 16)] = comp_d[pl.ds(SUP, 16)]
            return count - SUP, ws + 1

        def vec_body(i, carry):
            count, ws = carry
            dv = edt[pl.ds(i * 16, 16)]
            sv = est[pl.ds(i * 16, 16)]
            dl = dv - lo
            m = (dl >= 0) & (dl < C)
            mi = m.astype(jnp.int32)
            pos = count + plsc.cumsum(mi) - 1
            plsc.store_scatter(comp_s, [pos], sv, mask=m)
            plsc.store_scatter(comp_d, [pos], dl, mask=m)
            count = count + jnp.sum(mi)
            return lax.cond(count >= SUP, flush, lambda c: c, (count, ws))

        def blk_body(b, carry):
            pltpu.sync_copy(srcp.at[pl.ds(ebase + b * SB, SB)], est)
            pltpu.sync_copy(dstp.at[pl.ds(ebase + b * SB, SB)], edt)
            return lax.fori_loop(0, SB // 16, vec_body, carry)

        count, ws = lax.fori_loop(0, NBLK, blk_body,
                                  (jnp.int32(0), jnp.int32(0)))

        iot = lax.iota(jnp.int32, 16)

        def pad_body(j, _):
            off = j * 16
            m = (iot + off) < count
            comp_s[pl.ds(off, 16)] = jnp.where(m, comp_s[pl.ds(off, 16)], 0)
            comp_d[pl.ds(off, 16)] = jnp.where(m, comp_d[pl.ds(off, 16)], C)
            return 0

        lax.fori_loop(0, SUP // 16, pad_body, 0)
        pltpu.sync_copy(comp_s.at[pl.ds(0, SUP)],
                        lsrc.at[pl.ds(base + ws * SUP, SUP)])
        pltpu.sync_copy(comp_d.at[pl.ds(0, SUP)],
                        ldst.at[pl.ds(base + ws * SUP, SUP)])

        nblk = ws * 16 + lax.div(count + 127, 128)
        nbuf[...] = jnp.zeros((16,), jnp.int32) + nblk
        pltpu.sync_copy(nbuf, nblk1d.at[pl.ds(bucket * 16, 16)])


_route = pl.kernel(
    _route_body,
    out_type=(
        jax.ShapeDtypeStruct((NBUCKET * CAP,), jnp.int32),
        jax.ShapeDtypeStruct((NBUCKET * CAP,), jnp.int32),
        jax.ShapeDtypeStruct((NBUCKET * 16,), jnp.int32),
    ),
    mesh=_SC_MESH,
    scratch_types=dict(
        est=pltpu.VMEM((SB,), jnp.int32),
        edt=pltpu.VMEM((SB,), jnp.int32),
        comp_s=pltpu.VMEM((SUP + 128,), jnp.int32),
        comp_d=pltpu.VMEM((SUP + 128,), jnp.int32),
        nbuf=pltpu.VMEM((16,), jnp.int32),
    ),
    compiler_params=_SC_PARAMS,
)


def _stream_body(h, lsrc, ldst, nblk1d, z128, agg, *, acc, sbuf_s, sbuf_d,
                 idst0, idst1, rowbuf0, rowbuf1, nbuf,
                 semg0, semg1, sema0, sema1):
    cid = lax.axis_index("c")
    sid = lax.axis_index("s")

    def load_idst(kk, idst):
        for j in range(8):
            idst[pl.ds(j * 16, 16)] = sbuf_d[pl.ds(kk * 128 + j * 16, 16)]

    def gat(kk, rowbuf, semg):
        pltpu.async_copy(h.at[sbuf_s.at[pl.ds(kk * 128, 128)]], rowbuf, semg)

    for cc in range(CPC):
        chunk = cid * CPC + cc
        bucket = chunk * NS + sid
        base = bucket * CAP

        pltpu.sync_copy(z128, acc.at[pl.ds(sid * RPT, RPT)])
        plsc.subcore_barrier()

        pltpu.sync_copy(nblk1d.at[pl.ds(bucket * 16, 16)], nbuf)
        nb = jnp.max(nbuf[...])
        ns = lax.div(nb + 15, 16)

        def sup_body(s, _):
            pltpu.sync_copy(lsrc.at[pl.ds(base + s * SUP, SUP)], sbuf_s)
            pltpu.sync_copy(ldst.at[pl.ds(base + s * SUP, SUP)], sbuf_d)
            nbk = jnp.minimum(16, nb - s * 16)

            load_idst(0, idst0)
            gat(0, rowbuf0, semg0)

            def kb(k, _):
                even = lax.rem(k, 2) == 0

                @pl.when(even)
                def _():
                    @pl.when(k + 1 < nbk)
                    def _():
                        @pl.when(k >= 1)
                        def _():
                            pltpu.make_async_copy(
                                rowbuf1, acc.at[idst1], sema1).wait()
                        load_idst(k + 1, idst1)
                        gat(k + 1, rowbuf1, semg1)
                    pltpu.make_async_copy(
                        h.at[sbuf_s.at[pl.ds(k * 128, 128)]], rowbuf0,
                        semg0).wait()
                    pltpu.async_copy(rowbuf0, acc.at[idst0], sema0, add=True)

                @pl.when(jnp.logical_not(even))
                def _():
                    @pl.when(k + 1 < nbk)
                    def _():
                        pltpu.make_async_copy(
                            rowbuf0, acc.at[idst0], sema0).wait()
                        load_idst(k + 1, idst0)
                        gat(k + 1, rowbuf0, semg0)
                    pltpu.make_async_copy(
                        h.at[sbuf_s.at[pl.ds(k * 128, 128)]], rowbuf1,
                        semg1).wait()
                    pltpu.async_copy(rowbuf1, acc.at[idst1], sema1, add=True)

                return 0

            lax.fori_loop(0, nbk, kb, 0)

            last_even = lax.rem(nbk - 1, 2) == 0

            @pl.when(last_even)
            def _():
                pltpu.make_async_copy(rowbuf0, acc.at[idst0], sema0).wait()

                @pl.when(nbk > 1)
                def _():
                    pltpu.make_async_copy(rowbuf1, acc.at[idst1], sema1).wait()

            @pl.when(jnp.logical_not(last_even))
            def _():
                pltpu.make_async_copy(rowbuf1, acc.at[idst1], sema1).wait()
                pltpu.make_async_copy(rowbuf0, acc.at[idst0], sema0).wait()

            return 0

        lax.fori_loop(0, ns, sup_body, 0)
        plsc.subcore_barrier()

        pltpu.sync_copy(acc.at[pl.ds(sid * RPT, RPT)],
                        agg.at[pl.ds(chunk * C + sid * RPT, RPT)])
        plsc.subcore_barrier()


_stream = pl.kernel(
    _stream_body,
    out_type=(jax.ShapeDtypeStruct((NPAD, H), jnp.float32),),
    mesh=_SC_MESH,
    scratch_types=dict(
        acc=pltpu.VMEM_SHARED((ACC_ROWS, H), jnp.float32),
        sbuf_s=pltpu.VMEM((SUP,), jnp.int32),
        sbuf_d=pltpu.VMEM((SUP,), jnp.int32),
        idst0=pltpu.VMEM((128,), jnp.int32),
        idst1=pltpu.VMEM((128,), jnp.int32),
        rowbuf0=pltpu.VMEM((128, H), jnp.float32),
        rowbuf1=pltpu.VMEM((128, H), jnp.float32),
        nbuf=pltpu.VMEM((16,), jnp.int32),
        semg0=pltpu.SemaphoreType.DMA,
        semg1=pltpu.SemaphoreType.DMA,
        sema0=pltpu.SemaphoreType.DMA,
        sema1=pltpu.SemaphoreType.DMA,
    ),
    compiler_params=_SC_PARAMS,
)


def _ones_body(ldst, onesH, nblk1d, z128, cnt, *, acc, sbuf_d,
               idst0, idst1, onesbuf, nbuf, sema0, sema1):
    cid = lax.axis_index("c")
    sid = lax.axis_index("s")
    pltpu.sync_copy(onesH, onesbuf)

    def load_idst(kk, idst):
        for j in range(8):
            idst[pl.ds(j * 16, 16)] = sbuf_d[pl.ds(kk * 128 + j * 16, 16)]

    for cc in range(CPC):
        chunk = cid * CPC + cc
        bucket = chunk * NS + sid
        base = bucket * CAP

        pltpu.sync_copy(z128, acc.at[pl.ds(sid * RPT, RPT)])
        plsc.subcore_barrier()

        pltpu.sync_copy(nblk1d.at[pl.ds(bucket * 16, 16)], nbuf)
        nb = jnp.max(nbuf[...])
        ns = lax.div(nb + 15, 16)

        def sup_body(s, _):
            pltpu.sync_copy(ldst.at[pl.ds(base + s * SUP, SUP)], sbuf_d)
            nbk = jnp.minimum(16, nb - s * 16)

            def kb(k, _):
                even = lax.rem(k, 2) == 0

                @pl.when(even)
                def _():
                    @pl.when(k >= 2)
                    def _():
                        pltpu.make_async_copy(
                            onesbuf, acc.at[idst0], sema0).wait()
                    load_idst(k, idst0)
                    pltpu.async_copy(onesbuf, acc.at[idst0], sema0, add=True)

                @pl.when(jnp.logical_not(even))
                def _():
                    @pl.when(k >= 2)
                    def _():
                        pltpu.make_async_copy(
                            onesbuf, acc.at[idst1], sema1).wait()
                    load_idst(k, idst1)
                    pltpu.async_copy(onesbuf, acc.at[idst1], sema1, add=True)

                return 0

            lax.fori_loop(0, nbk, kb, 0)

            last_even = lax.rem(nbk - 1, 2) == 0

            @pl.when(last_even)
            def _():
                pltpu.make_async_copy(onesbuf, acc.at[idst0], sema0).wait()

                @pl.when(nbk > 1)
                def _():
                    pltpu.make_async_copy(onesbuf, acc.at[idst1], sema1).wait()

            @pl.when(jnp.logical_not(last_even))
            def _():
                pltpu.make_async_copy(onesbuf, acc.at[idst1], sema1).wait()
                pltpu.make_async_copy(onesbuf, acc.at[idst0], sema0).wait()

            return 0

        lax.fori_loop(0, ns, sup_body, 0)
        plsc.subcore_barrier()

        pltpu.sync_copy(acc.at[pl.ds(sid * RPT, RPT)],
                        cnt.at[pl.ds(chunk * C + sid * RPT, RPT)])
        plsc.subcore_barrier()


_cnt_stream = pl.kernel(
    _ones_body,
    out_type=(jax.ShapeDtypeStruct((NPAD, H), jnp.float32),),
    mesh=_SC_MESH,
    scratch_types=dict(
        acc=pltpu.VMEM_SHARED((ACC_ROWS, H), jnp.float32),
        sbuf_d=pltpu.VMEM((SUP,), jnp.int32),
        idst0=pltpu.VMEM((128,), jnp.int32),
        idst1=pltpu.VMEM((128,), jnp.int32),
        onesbuf=pltpu.VMEM((128, H), jnp.float32),
        nbuf=pltpu.VMEM((16,), jnp.int32),
        sema0=pltpu.SemaphoreType.DMA,
        sema1=pltpu.SemaphoreType.DMA,
    ),
    compiler_params=_SC_PARAMS,
)


def kernel(x, edge_index, W_in, b_in, W_l_0, b_l_0, W_r_0, ln_g_0, ln_b_0,
           W_l_1, b_l_1, W_r_1, ln_g_1, ln_b_1,
           W_l_2, b_l_2, W_r_2, ln_g_2, ln_b_2):
    src = edge_index[0]
    dst = edge_index[1]
    srcp = jnp.concatenate([src, jnp.zeros((E_PAD - E,), jnp.int32)])
    dstp = jnp.concatenate(
        [dst, jnp.full((E_PAD - E,), PAD_DST, jnp.int32)])
    z128 = jnp.zeros((RPT, H), jnp.float32)

    h = _proj(x, W_in, b_in)
    lsrc, ldst, nblk1d = _route(srcp, dstp)
    onesH = jnp.ones((128, H), jnp.float32)
    (cnt,) = _cnt_stream(ldst, onesH, nblk1d, z128)
    layers = [
        (W_l_0, b_l_0, W_r_0, ln_g_0, ln_b_0),
        (W_l_1, b_l_1, W_r_1, ln_g_1, ln_b_1),
        (W_l_2, b_l_2, W_r_2, ln_g_2, ln_b_2),
    ]
    for (W_l, b_l, W_r, g, b) in layers:
        (agg,) = _stream(h, lsrc, ldst, nblk1d, z128)
        h = _layer_update(h, agg, cnt, W_l, b_l, W_r, g, b)
    return h


# speedup vs baseline: 4.7624x; 1.0146x over previous
import functools

import jax
import jax.numpy as jnp
from jax import lax
from jax.experimental import pallas as pl
from jax.experimental.pallas import tpu as pltpu
from jax.experimental.pallas import tpu_sc as plsc

N = 50000
E = 800000
D_IN = 64
H = 128
EPS = 1e-5
BN = 400

NCHUNK = 6
CPC = 3
C = 8448
NPAD = NCHUNK * C
ACC_ROWS = C + 16
RPT = C // 16
NS = 16
SB = 2048
NBLK = 26
EPT = SB * NBLK
E_PAD = NS * EPT
PAD_DST = 1 << 20
SUP = 4096
SBLK = SUP // 128
CAP = EPT + SUP
NBUCKET = NCHUNK * NS

_SC_PARAMS = pltpu.CompilerParams(needs_layout_passes=False)
_SC_MESH = plsc.VectorSubcoreMesh(core_axis_name="c", subcore_axis_name="s")


def _proj_body(x_ref, w_ref, b_ref, o_ref):
    o_ref[...] = jax.nn.relu(
        jnp.dot(x_ref[...], w_ref[...], preferred_element_type=jnp.float32)
        + b_ref[...]
    )


def _proj(x, W_in, b_in):
    return pl.pallas_call(
        _proj_body,
        grid=(N // BN,),
        in_specs=[
            pl.BlockSpec((BN, D_IN), lambda i: (i, 0)),
            pl.BlockSpec((D_IN, H), lambda i: (0, 0)),
            pl.BlockSpec((1, H), lambda i: (0, 0)),
        ],
        out_specs=pl.BlockSpec((BN, H), lambda i: (i, 0)),
        out_shape=jax.ShapeDtypeStruct((N, H), jnp.float32),
    )(x, W_in, b_in.reshape(1, H))


def _layer_body(h_ref, agg_ref, cnt_ref, wl_ref, bl_ref, wr_ref, g_ref, b_ref,
                o_ref):
    h = h_ref[...]
    inv = 1.0 / jnp.maximum(cnt_ref[...][:, 0:1], 1.0)
    mean = agg_ref[...] * inv
    t = (jnp.dot(mean, wl_ref[...], preferred_element_type=jnp.float32)
         + bl_ref[...]
         + jnp.dot(h, wr_ref[...], preferred_element_type=jnp.float32))
    mu = jnp.mean(t, axis=-1, keepdims=True)
    var = jnp.mean((t - mu) ** 2, axis=-1, keepdims=True)
    y = (t - mu) * lax.rsqrt(var + EPS) * g_ref[...] + b_ref[...]
    o_ref[...] = jax.nn.relu(y) + h


def _layer_update(h, agg, cnt, W_l, b_l, W_r, g, b):
    return pl.pallas_call(
        _layer_body,
        grid=(N // BN,),
        in_specs=[
            pl.BlockSpec((BN, H), lambda i: (i, 0)),
            pl.BlockSpec((BN, H), lambda i: (i, 0)),
            pl.BlockSpec((BN, H), lambda i: (i, 0)),
            pl.BlockSpec((H, H), lambda i: (0, 0)),
            pl.BlockSpec((1, H), lambda i: (0, 0)),
            pl.BlockSpec((H, H), lambda i: (0, 0)),
            pl.BlockSpec((1, H), lambda i: (0, 0)),
            pl.BlockSpec((1, H), lambda i: (0, 0)),
        ],
        out_specs=pl.BlockSpec((BN, H), lambda i: (i, 0)),
        out_shape=jax.ShapeDtypeStruct((N, H), jnp.float32),
    )(h, agg, cnt, W_l, b_l.reshape(1, H), W_r, g.reshape(1, H),
      b.reshape(1, H))


def _route_body(srcp, dstp, lsrc, ldst, nblk1d, *, est, edt, comp_s, comp_d,
                nbuf):
    cid = lax.axis_index("c")
    sid = lax.axis_index("s")
    ebase = sid * EPT

    for cc in range(CPC):
        chunk = cid * CPC + cc
        lo = chunk * C
        bucket = chunk * NS + sid
        base = bucket * CAP

        def flush(carry):
            count, ws = carry
            pltpu.sync_copy(comp_s.at[pl.ds(0, SUP)],
                            lsrc.at[pl.ds(base + ws * SUP, SUP)])
            pltpu.sync_copy(comp_d.at[pl.ds(0, SUP)],
                            ldst.at[pl.ds(base + ws * SUP, SUP)])
            comp_s[pl.ds(0, 16)] = comp_s[pl.ds(SUP, 16)]
            comp_d[pl.ds(0, 16)] = comp_d[pl.ds(SUP, 16)]
            return count - SUP, ws + 1

        def vec_body(i, carry):
            count, ws = carry
            dv = edt[pl.ds(i * 16, 16)]
            sv = est[pl.ds(i * 16, 16)]
            dl = dv - lo
            m = (dl >= 0) & (dl < C)
            mi = m.astype(jnp.int32)
            pos = count + plsc.cumsum(mi) - 1
            plsc.store_scatter(comp_s, [pos], sv, mask=m)
            plsc.store_scatter(comp_d, [pos], dl, mask=m)
            count = count + jnp.sum(mi)
            return lax.cond(count >= SUP, flush, lambda c: c, (count, ws))

        def blk_body(b, carry):
            pltpu.sync_copy(srcp.at[pl.ds(ebase + b * SB, SB)], est)
            pltpu.sync_copy(dstp.at[pl.ds(ebase + b * SB, SB)], edt)
            return lax.fori_loop(0, SB // 16, vec_body, carry)

        count, ws = lax.fori_loop(0, NBLK, blk_body,
                                  (jnp.int32(0), jnp.int32(0)))

        iot = lax.iota(jnp.int32, 16)

        def pad_body(j, _):
            off = j * 16
            m = (iot + off) < count
            comp_s[pl.ds(off, 16)] = jnp.where(m, comp_s[pl.ds(off, 16)], 0)
            comp_d[pl.ds(off, 16)] = jnp.where(m, comp_d[pl.ds(off, 16)], C)
            return 0

        lax.fori_loop(0, SUP // 16, pad_body, 0)
        pltpu.sync_copy(comp_s.at[pl.ds(0, SUP)],
                        lsrc.at[pl.ds(base + ws * SUP, SUP)])
        pltpu.sync_copy(comp_d.at[pl.ds(0, SUP)],
                        ldst.at[pl.ds(base + ws * SUP, SUP)])

        nblk = ws * SBLK + lax.div(count + 127, 128)
        nbuf[...] = jnp.zeros((16,), jnp.int32) + nblk
        pltpu.sync_copy(nbuf, nblk1d.at[pl.ds(bucket * 16, 16)])


_route = pl.kernel(
    _route_body,
    out_type=(
        jax.ShapeDtypeStruct((NBUCKET * CAP,), jnp.int32),
        jax.ShapeDtypeStruct((NBUCKET * CAP,), jnp.int32),
        jax.ShapeDtypeStruct((NBUCKET * 16,), jnp.int32),
    ),
    mesh=_SC_MESH,
    scratch_types=dict(
        est=pltpu.VMEM((SB,), jnp.int32),
        edt=pltpu.VMEM((SB,), jnp.int32),
        comp_s=pltpu.VMEM((SUP + 128,), jnp.int32),
        comp_d=pltpu.VMEM((SUP + 128,), jnp.int32),
        nbuf=pltpu.VMEM((16,), jnp.int32),
    ),
    compiler_params=_SC_PARAMS,
)


def _stream_body(h, lsrc, ldst, nblk1d, z128, agg, *, acc, sbuf_s, sbuf_d,
                 idst0, idst1, rowbuf0, rowbuf1, nbuf,
                 semg0, semg1, sema0, sema1):
    cid = lax.axis_index("c")
    sid = lax.axis_index("s")

    def load_idst(kk, idst):
        for j in range(8):
            idst[pl.ds(j * 16, 16)] = sbuf_d[pl.ds(kk * 128 + j * 16, 16)]

    def gat(kk, rowbuf, semg):
        pltpu.async_copy(h.at[sbuf_s.at[pl.ds(kk * 128, 128)]], rowbuf, semg)

    for cc in range(CPC):
        chunk = cid * CPC + cc
        bucket = chunk * NS + sid
        base = bucket * CAP

        pltpu.sync_copy(z128, acc.at[pl.ds(sid * RPT, RPT)])
        plsc.subcore_barrier()

        pltpu.sync_copy(nblk1d.at[pl.ds(bucket * 16, 16)], nbuf)
        nb = jnp.max(nbuf[...])
        ns = lax.div(nb + SBLK - 1, SBLK)

        def sup_body(s, _):
            pltpu.sync_copy(lsrc.at[pl.ds(base + s * SUP, SUP)], sbuf_s)
            pltpu.sync_copy(ldst.at[pl.ds(base + s * SUP, SUP)], sbuf_d)
            nbk = jnp.minimum(SBLK, nb - s * SBLK)

            load_idst(0, idst0)
            gat(0, rowbuf0, semg0)

            def kb(k, _):
                even = lax.rem(k, 2) == 0

                @pl.when(even)
                def _():
                    @pl.when(k + 1 < nbk)
                    def _():
                        @pl.when(k >= 1)
                        def _():
                            pltpu.make_async_copy(
                                rowbuf1, acc.at[idst1], sema1).wait()
                        load_idst(k + 1, idst1)
                        gat(k + 1, rowbuf1, semg1)
                    pltpu.make_async_copy(
                        h.at[sbuf_s.at[pl.ds(k * 128, 128)]], rowbuf0,
                        semg0).wait()
                    pltpu.async_copy(rowbuf0, acc.at[idst0], sema0, add=True)

                @pl.when(jnp.logical_not(even))
                def _():
                    @pl.when(k + 1 < nbk)
                    def _():
                        pltpu.make_async_copy(
                            rowbuf0, acc.at[idst0], sema0).wait()
                        load_idst(k + 1, idst0)
                        gat(k + 1, rowbuf0, semg0)
                    pltpu.make_async_copy(
                        h.at[sbuf_s.at[pl.ds(k * 128, 128)]], rowbuf1,
                        semg1).wait()
                    pltpu.async_copy(rowbuf1, acc.at[idst1], sema1, add=True)

                return 0

            lax.fori_loop(0, nbk, kb, 0)

            last_even = lax.rem(nbk - 1, 2) == 0

            @pl.when(last_even)
            def _():
                pltpu.make_async_copy(rowbuf0, acc.at[idst0], sema0).wait()

                @pl.when(nbk > 1)
                def _():
                    pltpu.make_async_copy(rowbuf1, acc.at[idst1], sema1).wait()

            @pl.when(jnp.logical_not(last_even))
            def _():
                pltpu.make_async_copy(rowbuf1, acc.at[idst1], sema1).wait()
                pltpu.make_async_copy(rowbuf0, acc.at[idst0], sema0).wait()

            return 0

        lax.fori_loop(0, ns, sup_body, 0)
        plsc.subcore_barrier()

        pltpu.sync_copy(acc.at[pl.ds(sid * RPT, RPT)],
                        agg.at[pl.ds(chunk * C + sid * RPT, RPT)])
        plsc.subcore_barrier()


_stream = pl.kernel(
    _stream_body,
    out_type=(jax.ShapeDtypeStruct((NPAD, H), jnp.float32),),
    mesh=_SC_MESH,
    scratch_types=dict(
        acc=pltpu.VMEM_SHARED((ACC_ROWS, H), jnp.float32),
        sbuf_s=pltpu.VMEM((SUP,), jnp.int32),
        sbuf_d=pltpu.VMEM((SUP,), jnp.int32),
        idst0=pltpu.VMEM((128,), jnp.int32),
        idst1=pltpu.VMEM((128,), jnp.int32),
        rowbuf0=pltpu.VMEM((128, H), jnp.float32),
        rowbuf1=pltpu.VMEM((128, H), jnp.float32),
        nbuf=pltpu.VMEM((16,), jnp.int32),
        semg0=pltpu.SemaphoreType.DMA,
        semg1=pltpu.SemaphoreType.DMA,
        sema0=pltpu.SemaphoreType.DMA,
        sema1=pltpu.SemaphoreType.DMA,
    ),
    compiler_params=_SC_PARAMS,
)


def _ones_body(ldst, onesH, nblk1d, z128, cnt, *, acc, sbuf_d,
               idst0, idst1, onesbuf, nbuf, sema0, sema1):
    cid = lax.axis_index("c")
    sid = lax.axis_index("s")
    pltpu.sync_copy(onesH, onesbuf)

    def load_idst(kk, idst):
        for j in range(8):
            idst[pl.ds(j * 16, 16)] = sbuf_d[pl.ds(kk * 128 + j * 16, 16)]

    for cc in range(CPC):
        chunk = cid * CPC + cc
        bucket = chunk * NS + sid
        base = bucket * CAP

        pltpu.sync_copy(z128, acc.at[pl.ds(sid * RPT, RPT)])
        plsc.subcore_barrier()

        pltpu.sync_copy(nblk1d.at[pl.ds(bucket * 16, 16)], nbuf)
        nb = jnp.max(nbuf[...])
        ns = lax.div(nb + SBLK - 1, SBLK)

        def sup_body(s, _):
            pltpu.sync_copy(ldst.at[pl.ds(base + s * SUP, SUP)], sbuf_d)
            nbk = jnp.minimum(SBLK, nb - s * SBLK)

            def kb(k, _):
                even = lax.rem(k, 2) == 0

                @pl.when(even)
                def _():
                    @pl.when(k >= 2)
                    def _():
                        pltpu.make_async_copy(
                            onesbuf, acc.at[idst0], sema0).wait()
                    load_idst(k, idst0)
                    pltpu.async_copy(onesbuf, acc.at[idst0], sema0, add=True)

                @pl.when(jnp.logical_not(even))
                def _():
                    @pl.when(k >= 2)
                    def _():
                        pltpu.make_async_copy(
                            onesbuf, acc.at[idst1], sema1).wait()
                    load_idst(k, idst1)
                    pltpu.async_copy(onesbuf, acc.at[idst1], sema1, add=True)

                return 0

            lax.fori_loop(0, nbk, kb, 0)

            last_even = lax.rem(nbk - 1, 2) == 0

            @pl.when(last_even)
            def _():
                pltpu.make_async_copy(onesbuf, acc.at[idst0], sema0).wait()

                @pl.when(nbk > 1)
                def _():
                    pltpu.make_async_copy(onesbuf, acc.at[idst1], sema1).wait()

            @pl.when(jnp.logical_not(last_even))
            def _():
                pltpu.make_async_copy(onesbuf, acc.at[idst1], sema1).wait()
                pltpu.make_async_copy(onesbuf, acc.at[idst0], sema0).wait()

            return 0

        lax.fori_loop(0, ns, sup_body, 0)
        plsc.subcore_barrier()

        pltpu.sync_copy(acc.at[pl.ds(sid * RPT, RPT)],
                        cnt.at[pl.ds(chunk * C + sid * RPT, RPT)])
        plsc.subcore_barrier()


_cnt_stream = pl.kernel(
    _ones_body,
    out_type=(jax.ShapeDtypeStruct((NPAD, H), jnp.float32),),
    mesh=_SC_MESH,
    scratch_types=dict(
        acc=pltpu.VMEM_SHARED((ACC_ROWS, H), jnp.float32),
        sbuf_d=pltpu.VMEM((SUP,), jnp.int32),
        idst0=pltpu.VMEM((128,), jnp.int32),
        idst1=pltpu.VMEM((128,), jnp.int32),
        onesbuf=pltpu.VMEM((128, H), jnp.float32),
        nbuf=pltpu.VMEM((16,), jnp.int32),
        sema0=pltpu.SemaphoreType.DMA,
        sema1=pltpu.SemaphoreType.DMA,
    ),
    compiler_params=_SC_PARAMS,
)


def kernel(x, edge_index, W_in, b_in, W_l_0, b_l_0, W_r_0, ln_g_0, ln_b_0,
           W_l_1, b_l_1, W_r_1, ln_g_1, ln_b_1,
           W_l_2, b_l_2, W_r_2, ln_g_2, ln_b_2):
    src = edge_index[0]
    dst = edge_index[1]
    srcp = jnp.concatenate([src, jnp.zeros((E_PAD - E,), jnp.int32)])
    dstp = jnp.concatenate(
        [dst, jnp.full((E_PAD - E,), PAD_DST, jnp.int32)])
    z128 = jnp.zeros((RPT, H), jnp.float32)

    h = _proj(x, W_in, b_in)
    lsrc, ldst, nblk1d = _route(srcp, dstp)
    onesH = jnp.ones((128, H), jnp.float32)
    (cnt,) = _cnt_stream(ldst, onesH, nblk1d, z128)
    layers = [
        (W_l_0, b_l_0, W_r_0, ln_g_0, ln_b_0),
        (W_l_1, b_l_1, W_r_1, ln_g_1, ln_b_1),
        (W_l_2, b_l_2, W_r_2, ln_g_2, ln_b_2),
    ]
    for (W_l, b_l, W_r, g, b) in layers:
        (agg,) = _stream(h, lsrc, ldst, nblk1d, z128)
        h = _layer_update(h, agg, cnt, W_l, b_l, W_r, g, b)
    return h


# speedup vs baseline: 4.9588x; 1.0412x over previous
import functools

import jax
import jax.numpy as jnp
from jax import lax
from jax.experimental import pallas as pl
from jax.experimental.pallas import tpu as pltpu
from jax.experimental.pallas import tpu_sc as plsc

N = 50000
E = 800000
D_IN = 64
H = 128
EPS = 1e-5
BN = 400

NCHUNK = 6
CPC = 3
C = 8448
NPAD = NCHUNK * C
ACC_ROWS = C + 16
RPT = C // 16
NS = 16
SB = 2048
NBLK = 26
EPT = SB * NBLK
E_PAD = NS * EPT
PAD_DST = 1 << 20
SUP = 4096
SBLK = SUP // 128
CAP = EPT + SUP
NBUCKET = NCHUNK * NS

_SC_PARAMS = pltpu.CompilerParams(needs_layout_passes=False)
_SC_MESH = plsc.VectorSubcoreMesh(core_axis_name="c", subcore_axis_name="s")


def _proj_body(x_ref, w_ref, b_ref, o_ref):
    o_ref[...] = jax.nn.relu(
        jnp.dot(x_ref[...], w_ref[...], preferred_element_type=jnp.float32)
        + b_ref[...]
    )


def _proj(x, W_in, b_in):
    return pl.pallas_call(
        _proj_body,
        grid=(N // BN,),
        in_specs=[
            pl.BlockSpec((BN, D_IN), lambda i: (i, 0)),
            pl.BlockSpec((D_IN, H), lambda i: (0, 0)),
            pl.BlockSpec((1, H), lambda i: (0, 0)),
        ],
        out_specs=pl.BlockSpec((BN, H), lambda i: (i, 0)),
        out_shape=jax.ShapeDtypeStruct((N, H), jnp.float32),
    )(x, W_in, b_in.reshape(1, H))


def _layer_body(h_ref, agg_ref, cnt_ref, wl_ref, bl_ref, wr_ref, g_ref, b_ref,
                o_ref):
    h = h_ref[...]
    inv = 1.0 / jnp.maximum(cnt_ref[...][:, 0:1], 1.0)
    mean = agg_ref[...] * inv
    t = (jnp.dot(mean, wl_ref[...], preferred_element_type=jnp.float32)
         + bl_ref[...]
         + jnp.dot(h, wr_ref[...], preferred_element_type=jnp.float32))
    mu = jnp.mean(t, axis=-1, keepdims=True)
    var = jnp.mean((t - mu) ** 2, axis=-1, keepdims=True)
    y = (t - mu) * lax.rsqrt(var + EPS) * g_ref[...] + b_ref[...]
    o_ref[...] = jax.nn.relu(y) + h


def _layer_update(h, agg, cnt, W_l, b_l, W_r, g, b):
    return pl.pallas_call(
        _layer_body,
        grid=(N // BN,),
        in_specs=[
            pl.BlockSpec((BN, H), lambda i: (i, 0)),
            pl.BlockSpec((BN, H), lambda i: (i, 0)),
            pl.BlockSpec((BN, H), lambda i: (i, 0)),
            pl.BlockSpec((H, H), lambda i: (0, 0)),
            pl.BlockSpec((1, H), lambda i: (0, 0)),
            pl.BlockSpec((H, H), lambda i: (0, 0)),
            pl.BlockSpec((1, H), lambda i: (0, 0)),
            pl.BlockSpec((1, H), lambda i: (0, 0)),
        ],
        out_specs=pl.BlockSpec((BN, H), lambda i: (i, 0)),
        out_shape=jax.ShapeDtypeStruct((N, H), jnp.float32),
    )(h, agg, cnt, W_l, b_l.reshape(1, H), W_r, g.reshape(1, H),
      b.reshape(1, H))


def _route_body(srcp, dstp, lsrc, ldst, nblk1d, *, est0, edt0, est1, edt1,
                comp_s, comp_d, nbuf, ss0, ss1):
    cid = lax.axis_index("c")
    sid = lax.axis_index("s")
    ebase = sid * EPT

    def load_stage(b, est, edt, sem):
        pltpu.async_copy(srcp.at[pl.ds(ebase + b * SB, SB)], est, sem)
        pltpu.async_copy(dstp.at[pl.ds(ebase + b * SB, SB)], edt, sem)

    def wait_stage(est, edt, sem):
        pltpu.make_async_copy(srcp.at[pl.ds(0, SB)], est, sem).wait()
        pltpu.make_async_copy(dstp.at[pl.ds(0, SB)], edt, sem).wait()

    for cc in range(CPC):
        chunk = cid * CPC + cc
        lo = chunk * C
        bucket = chunk * NS + sid
        base = bucket * CAP

        def flush(carry):
            count, ws = carry
            pltpu.sync_copy(comp_s.at[pl.ds(0, SUP)],
                            lsrc.at[pl.ds(base + ws * SUP, SUP)])
            pltpu.sync_copy(comp_d.at[pl.ds(0, SUP)],
                            ldst.at[pl.ds(base + ws * SUP, SUP)])
            comp_s[pl.ds(0, 16)] = comp_s[pl.ds(SUP, 16)]
            comp_d[pl.ds(0, 16)] = comp_d[pl.ds(SUP, 16)]
            return count - SUP, ws + 1

        def make_vec_body(est, edt):
            def vec_body(i, carry):
                count, ws = carry
                dv = edt[pl.ds(i * 16, 16)]
                sv = est[pl.ds(i * 16, 16)]
                dl = dv - lo
                m = (dl >= 0) & (dl < C)
                mi = m.astype(jnp.int32)
                pos = count + plsc.cumsum(mi) - 1
                plsc.store_scatter(comp_s, [pos], sv, mask=m)
                plsc.store_scatter(comp_d, [pos], dl, mask=m)
                count = count + jnp.sum(mi)
                return lax.cond(count >= SUP, flush, lambda c: c, (count, ws))
            return vec_body

        vb0 = make_vec_body(est0, edt0)
        vb1 = make_vec_body(est1, edt1)

        load_stage(0, est0, edt0, ss0)

        def pair_body(p, carry):
            load_stage(2 * p + 1, est1, edt1, ss1)
            wait_stage(est0, edt0, ss0)
            carry = lax.fori_loop(0, SB // 16, vb0, carry)

            @pl.when(p < NBLK // 2 - 1)
            def _():
                load_stage(2 * p + 2, est0, edt0, ss0)

            wait_stage(est1, edt1, ss1)
            return lax.fori_loop(0, SB // 16, vb1, carry)

        count, ws = lax.fori_loop(0, NBLK // 2, pair_body,
                                  (jnp.int32(0), jnp.int32(0)))

        iot = lax.iota(jnp.int32, 16)

        def pad_body(j, _):
            off = j * 16
            m = (iot + off) < count
            comp_s[pl.ds(off, 16)] = jnp.where(m, comp_s[pl.ds(off, 16)], 0)
            comp_d[pl.ds(off, 16)] = jnp.where(m, comp_d[pl.ds(off, 16)], C)
            return 0

        lax.fori_loop(0, SUP // 16, pad_body, 0)
        pltpu.sync_copy(comp_s.at[pl.ds(0, SUP)],
                        lsrc.at[pl.ds(base + ws * SUP, SUP)])
        pltpu.sync_copy(comp_d.at[pl.ds(0, SUP)],
                        ldst.at[pl.ds(base + ws * SUP, SUP)])

        nblk = ws * SBLK + lax.div(count + 127, 128)
        nbuf[...] = jnp.zeros((16,), jnp.int32) + nblk
        pltpu.sync_copy(nbuf, nblk1d.at[pl.ds(bucket * 16, 16)])


_route = pl.kernel(
    _route_body,
    out_type=(
        jax.ShapeDtypeStruct((NBUCKET * CAP,), jnp.int32),
        jax.ShapeDtypeStruct((NBUCKET * CAP,), jnp.int32),
        jax.ShapeDtypeStruct((NBUCKET * 16,), jnp.int32),
    ),
    mesh=_SC_MESH,
    scratch_types=dict(
        est0=pltpu.VMEM((SB,), jnp.int32),
        edt0=pltpu.VMEM((SB,), jnp.int32),
        est1=pltpu.VMEM((SB,), jnp.int32),
        edt1=pltpu.VMEM((SB,), jnp.int32),
        comp_s=pltpu.VMEM((SUP + 128,), jnp.int32),
        comp_d=pltpu.VMEM((SUP + 128,), jnp.int32),
        nbuf=pltpu.VMEM((16,), jnp.int32),
        ss0=pltpu.SemaphoreType.DMA,
        ss1=pltpu.SemaphoreType.DMA,
    ),
    compiler_params=_SC_PARAMS,
)


def _stream_body(h, lsrc, ldst, nblk1d, z128, agg, *, acc, sbuf_s, sbuf_d,
                 idst0, idst1, rowbuf0, rowbuf1, nbuf,
                 semg0, semg1, sema0, sema1):
    cid = lax.axis_index("c")
    sid = lax.axis_index("s")

    def load_idst(kk, idst):
        for j in range(8):
            idst[pl.ds(j * 16, 16)] = sbuf_d[pl.ds(kk * 128 + j * 16, 16)]

    def gat(kk, rowbuf, semg):
        pltpu.async_copy(h.at[sbuf_s.at[pl.ds(kk * 128, 128)]], rowbuf, semg)

    for cc in range(CPC):
        chunk = cid * CPC + cc
        bucket = chunk * NS + sid
        base = bucket * CAP

        pltpu.sync_copy(z128, acc.at[pl.ds(sid * RPT, RPT)])
        plsc.subcore_barrier()

        pltpu.sync_copy(nblk1d.at[pl.ds(bucket * 16, 16)], nbuf)
        nb = jnp.max(nbuf[...])
        ns = lax.div(nb + SBLK - 1, SBLK)

        def sup_body(s, _):
            pltpu.sync_copy(lsrc.at[pl.ds(base + s * SUP, SUP)], sbuf_s)
            pltpu.sync_copy(ldst.at[pl.ds(base + s * SUP, SUP)], sbuf_d)
            nbk = jnp.minimum(SBLK, nb - s * SBLK)

            load_idst(0, idst0)
            gat(0, rowbuf0, semg0)

            def kb(k, _):
                even = lax.rem(k, 2) == 0

                @pl.when(even)
                def _():
                    @pl.when(k + 1 < nbk)
                    def _():
                        @pl.when(k >= 1)
                        def _():
                            pltpu.make_async_copy(
                                rowbuf1, acc.at[idst1], sema1).wait()
                        load_idst(k + 1, idst1)
                        gat(k + 1, rowbuf1, semg1)
                    pltpu.make_async_copy(
                        h.at[sbuf_s.at[pl.ds(k * 128, 128)]], rowbuf0,
                        semg0).wait()
                    pltpu.async_copy(rowbuf0, acc.at[idst0], sema0, add=True)

                @pl.when(jnp.logical_not(even))
                def _():
                    @pl.when(k + 1 < nbk)
                    def _():
                        pltpu.make_async_copy(
                            rowbuf0, acc.at[idst0], sema0).wait()
                        load_idst(k + 1, idst0)
                        gat(k + 1, rowbuf0, semg0)
                    pltpu.make_async_copy(
                        h.at[sbuf_s.at[pl.ds(k * 128, 128)]], rowbuf1,
                        semg1).wait()
                    pltpu.async_copy(rowbuf1, acc.at[idst1], sema1, add=True)

                return 0

            lax.fori_loop(0, nbk, kb, 0)

            last_even = lax.rem(nbk - 1, 2) == 0

            @pl.when(last_even)
            def _():
                pltpu.make_async_copy(rowbuf0, acc.at[idst0], sema0).wait()

                @pl.when(nbk > 1)
                def _():
                    pltpu.make_async_copy(rowbuf1, acc.at[idst1], sema1).wait()

            @pl.when(jnp.logical_not(last_even))
            def _():
                pltpu.make_async_copy(rowbuf1, acc.at[idst1], sema1).wait()
                pltpu.make_async_copy(rowbuf0, acc.at[idst0], sema0).wait()

            return 0

        lax.fori_loop(0, ns, sup_body, 0)
        plsc.subcore_barrier()

        pltpu.sync_copy(acc.at[pl.ds(sid * RPT, RPT)],
                        agg.at[pl.ds(chunk * C + sid * RPT, RPT)])
        plsc.subcore_barrier()


_stream = pl.kernel(
    _stream_body,
    out_type=(jax.ShapeDtypeStruct((NPAD, H), jnp.float32),),
    mesh=_SC_MESH,
    scratch_types=dict(
        acc=pltpu.VMEM_SHARED((ACC_ROWS, H), jnp.float32),
        sbuf_s=pltpu.VMEM((SUP,), jnp.int32),
        sbuf_d=pltpu.VMEM((SUP,), jnp.int32),
        idst0=pltpu.VMEM((128,), jnp.int32),
        idst1=pltpu.VMEM((128,), jnp.int32),
        rowbuf0=pltpu.VMEM((128, H), jnp.float32),
        rowbuf1=pltpu.VMEM((128, H), jnp.float32),
        nbuf=pltpu.VMEM((16,), jnp.int32),
        semg0=pltpu.SemaphoreType.DMA,
        semg1=pltpu.SemaphoreType.DMA,
        sema0=pltpu.SemaphoreType.DMA,
        sema1=pltpu.SemaphoreType.DMA,
    ),
    compiler_params=_SC_PARAMS,
)


def _ones_body(ldst, onesH, nblk1d, z128, cnt, *, acc, sbuf_d,
               idst0, idst1, onesbuf, nbuf, sema0, sema1):
    cid = lax.axis_index("c")
    sid = lax.axis_index("s")
    pltpu.sync_copy(onesH, onesbuf)

    def load_idst(kk, idst):
        for j in range(8):
            idst[pl.ds(j * 16, 16)] = sbuf_d[pl.ds(kk * 128 + j * 16, 16)]

    for cc in range(CPC):
        chunk = cid * CPC + cc
        bucket = chunk * NS + sid
        base = bucket * CAP

        pltpu.sync_copy(z128, acc.at[pl.ds(sid * RPT, RPT)])
        plsc.subcore_barrier()

        pltpu.sync_copy(nblk1d.at[pl.ds(bucket * 16, 16)], nbuf)
        nb = jnp.max(nbuf[...])
        ns = lax.div(nb + SBLK - 1, SBLK)

        def sup_body(s, _):
            pltpu.sync_copy(ldst.at[pl.ds(base + s * SUP, SUP)], sbuf_d)
            nbk = jnp.minimum(SBLK, nb - s * SBLK)

            def kb(k, _):
                even = lax.rem(k, 2) == 0

                @pl.when(even)
                def _():
                    @pl.when(k >= 2)
                    def _():
                        pltpu.make_async_copy(
                            onesbuf, acc.at[idst0], sema0).wait()
                    load_idst(k, idst0)
                    pltpu.async_copy(onesbuf, acc.at[idst0], sema0, add=True)

                @pl.when(jnp.logical_not(even))
                def _():
                    @pl.when(k >= 2)
                    def _():
                        pltpu.make_async_copy(
                            onesbuf, acc.at[idst1], sema1).wait()
                    load_idst(k, idst1)
                    pltpu.async_copy(onesbuf, acc.at[idst1], sema1, add=True)

                return 0

            lax.fori_loop(0, nbk, kb, 0)

            last_even = lax.rem(nbk - 1, 2) == 0

            @pl.when(last_even)
            def _():
                pltpu.make_async_copy(onesbuf, acc.at[idst0], sema0).wait()

                @pl.when(nbk > 1)
                def _():
                    pltpu.make_async_copy(onesbuf, acc.at[idst1], sema1).wait()

            @pl.when(jnp.logical_not(last_even))
            def _():
                pltpu.make_async_copy(onesbuf, acc.at[idst1], sema1).wait()
                pltpu.make_async_copy(onesbuf, acc.at[idst0], sema0).wait()

            return 0

        lax.fori_loop(0, ns, sup_body, 0)
        plsc.subcore_barrier()

        pltpu.sync_copy(acc.at[pl.ds(sid * RPT, RPT)],
                        cnt.at[pl.ds(chunk * C + sid * RPT, RPT)])
        plsc.subcore_barrier()


_cnt_stream = pl.kernel(
    _ones_body,
    out_type=(jax.ShapeDtypeStruct((NPAD, H), jnp.float32),),
    mesh=_SC_MESH,
    scratch_types=dict(
        acc=pltpu.VMEM_SHARED((ACC_ROWS, H), jnp.float32),
        sbuf_d=pltpu.VMEM((SUP,), jnp.int32),
        idst0=pltpu.VMEM((128,), jnp.int32),
        idst1=pltpu.VMEM((128,), jnp.int32),
        onesbuf=pltpu.VMEM((128, H), jnp.float32),
        nbuf=pltpu.VMEM((16,), jnp.int32),
        sema0=pltpu.SemaphoreType.DMA,
        sema1=pltpu.SemaphoreType.DMA,
    ),
    compiler_params=_SC_PARAMS,
)


def kernel(x, edge_index, W_in, b_in, W_l_0, b_l_0, W_r_0, ln_g_0, ln_b_0,
           W_l_1, b_l_1, W_r_1, ln_g_1, ln_b_1,
           W_l_2, b_l_2, W_r_2, ln_g_2, ln_b_2):
    src = edge_index[0]
    dst = edge_index[1]
    srcp = jnp.concatenate([src, jnp.zeros((E_PAD - E,), jnp.int32)])
    dstp = jnp.concatenate(
        [dst, jnp.full((E_PAD - E,), PAD_DST, jnp.int32)])
    z128 = jnp.zeros((RPT, H), jnp.float32)

    h = _proj(x, W_in, b_in)
    lsrc, ldst, nblk1d = _route(srcp, dstp)
    onesH = jnp.ones((128, H), jnp.float32)
    (cnt,) = _cnt_stream(ldst, onesH, nblk1d, z128)
    layers = [
        (W_l_0, b_l_0, W_r_0, ln_g_0, ln_b_0),
        (W_l_1, b_l_1, W_r_1, ln_g_1, ln_b_1),
        (W_l_2, b_l_2, W_r_2, ln_g_2, ln_b_2),
    ]
    for (W_l, b_l, W_r, g, b) in layers:
        (agg,) = _stream(h, lsrc, ldst, nblk1d, z128)
        h = _layer_update(h, agg, cnt, W_l, b_l, W_r, g, b)
    return h
